# baseline (speedup 1.0000x reference)
import functools

import jax
import jax.numpy as jnp
from jax import lax
from jax.experimental import pallas as pl
from jax.experimental.pallas import tpu as pltpu
from jax.experimental.pallas import tpu_sc as plsc

BATCH = 4
SEQ = 8192
D = 768
NLANE = D // 16

_NC = 2
_NS = 16
NW = _NC * _NS
ROWS_PER_W = SEQ // NW
C = 16
NCHUNKS = ROWS_PER_W // C

_mesh = plsc.VectorSubcoreMesh(core_axis_name="c", subcore_axis_name="s")


@functools.partial(
    pl.kernel,
    mesh=_mesh,
    out_type=jax.ShapeDtypeStruct((BATCH, SEQ, D), jnp.float32),
    compiler_params=pltpu.CompilerParams(
        disable_bounds_checks=True,
        disable_semaphore_checks=True,
        skip_device_barrier=True,
    ),
    scratch_types=[
        [pltpu.VMEM((C, D), jnp.float32) for _ in range(2)],
        [[pltpu.VMEM((C, D), jnp.float32) for _ in range(BATCH)]
         for _ in range(2)],
        [pltpu.SemaphoreType.DMA for _ in range(2)],
        [[pltpu.SemaphoreType.DMA for _ in range(BATCH)]
         for _ in range(2)],
        [[pltpu.SemaphoreType.DMA for _ in range(BATCH)]
         for _ in range(2)],
    ],
)
def _sc_pos_add(x_hbm, pos_hbm, out_hbm, pos_v, xb, sp, si, so):
    wid = lax.axis_index("s") * _NC + lax.axis_index("c")
    base = wid * ROWS_PER_W

    def rows(ci):
        return pl.ds(pl.multiple_of(base + ci * C, C), C)

    pltpu.async_copy(pos_hbm.at[rows(0)], pos_v[0], sp[0])
    for b in range(BATCH):
        pltpu.async_copy(x_hbm.at[b, rows(0)], xb[0][b], si[0][b])

    def outer(ci2, _):
        for ph in range(2):
            ci = ci2 * 2 + ph
            pos = pos_v[ph]

            def _drain_prev_outs():
                for b in range(BATCH):
                    pltpu.make_async_copy(
                        xb[1 - ph][b], out_hbm.at[0, rows(0)],
                        so[1 - ph][b]).wait()

            if ph == 0:
                pl.when(ci2 > 0)(_drain_prev_outs)
            else:
                _drain_prev_outs()

            @pl.when(ci < NCHUNKS - 1)
            def _():
                for b in range(BATCH):
                    pltpu.async_copy(x_hbm.at[b, rows(ci + 1)],
                                     xb[1 - ph][b], si[1 - ph][b])
                pltpu.async_copy(pos_hbm.at[rows(ci + 1)],
                                 pos_v[1 - ph], sp[1 - ph])

            pltpu.make_async_copy(pos_hbm.at[rows(0)], pos, sp[ph]).wait()
            for b in range(BATCH):
                pltpu.make_async_copy(
                    x_hbm.at[0, rows(0)], xb[ph][b], si[ph][b]).wait()

            @plsc.parallel_loop(0, C, unroll=3)
            def _(r):
                for j in range(NLANE):
                    s = pl.ds(j * 16, 16)
                    p = pos[r, s]
                    vals = [xb[ph][b][r, s] + p for b in range(BATCH)]
                    for b in range(BATCH):
                        xb[ph][b][r, s] = vals[b]

            for b in range(BATCH):
                pltpu.async_copy(xb[ph][b], out_hbm.at[b, rows(ci)],
                                 so[ph][b])
        return 0

    lax.fori_loop(0, NCHUNKS // 2, outer, 0)

    for b in range(BATCH):
        pltpu.make_async_copy(xb[1][b], out_hbm.at[0, rows(0)],
                              so[1][b]).wait()


def kernel(x, pos_table):
    return _sc_pos_add(x, pos_table)


# speedup vs baseline: 1.0337x; 1.0337x over previous
import functools

import jax
import jax.numpy as jnp
from jax import lax
from jax.experimental import pallas as pl
from jax.experimental.pallas import tpu as pltpu
from jax.experimental.pallas import tpu_sc as plsc

BATCH = 4
SEQ = 8192
D = 768
NLANE = D // 16

_NC = 2
_NS = 16
NW = _NC * _NS
ROWS_PER_W = SEQ // NW
C = 16
NCHUNKS = ROWS_PER_W // C

_mesh = plsc.VectorSubcoreMesh(core_axis_name="c", subcore_axis_name="s")


@functools.partial(
    pl.kernel,
    mesh=_mesh,
    out_type=jax.ShapeDtypeStruct((BATCH, SEQ, D), jnp.float32),
    compiler_params=pltpu.CompilerParams(
        disable_bounds_checks=True,
        disable_semaphore_checks=True,
        skip_device_barrier=True,
    ),
    scratch_types=[
        [pltpu.VMEM((C, D), jnp.float32) for _ in range(2)],
        [[pltpu.VMEM((C, D), jnp.float32) for _ in range(BATCH)]
         for _ in range(2)],
        [pltpu.SemaphoreType.DMA for _ in range(2)],
        [[pltpu.SemaphoreType.DMA for _ in range(BATCH)]
         for _ in range(2)],
        [[pltpu.SemaphoreType.DMA for _ in range(BATCH)]
         for _ in range(2)],
    ],
)
def _sc_pos_add(x_hbm, pos_hbm, out_hbm, pos_v, xb, sp, si, so):
    wid = lax.axis_index("s") * _NC + lax.axis_index("c")
    base = wid * ROWS_PER_W

    def rows(ci):
        return pl.ds(pl.multiple_of(base + ci * C, C), C)

    pltpu.async_copy(pos_hbm.at[rows(0)], pos_v[0], sp[0])
    for b in range(BATCH):
        pltpu.async_copy(x_hbm.at[b, rows(0)], xb[0][b], si[0][b])

    def outer(ci2, _):
        for ph in range(2):
            ci = ci2 * 2 + ph
            pos = pos_v[ph]

            def _drain_prev_outs():
                for b in range(BATCH):
                    pltpu.make_async_copy(
                        xb[1 - ph][b], out_hbm.at[0, rows(0)],
                        so[1 - ph][b]).wait()

            if ph == 0:
                pl.when(ci2 > 0)(_drain_prev_outs)
            else:
                _drain_prev_outs()

            @pl.when(ci < NCHUNKS - 1)
            def _():
                for b in range(BATCH):
                    pltpu.async_copy(x_hbm.at[b, rows(ci + 1)],
                                     xb[1 - ph][b], si[1 - ph][b])
                pltpu.async_copy(pos_hbm.at[rows(ci + 1)],
                                 pos_v[1 - ph], sp[1 - ph])

            pltpu.make_async_copy(pos_hbm.at[rows(0)], pos, sp[ph]).wait()
            for b in range(BATCH):
                pltpu.make_async_copy(
                    x_hbm.at[0, rows(0)], xb[ph][b], si[ph][b]).wait()

            @plsc.parallel_loop(0, C, unroll=2)
            def _(r):
                for j in range(NLANE):
                    s = pl.ds(j * 16, 16)
                    p = pos[r, s]
                    vals = [xb[ph][b][r, s] + p for b in range(BATCH)]
                    for b in range(BATCH):
                        xb[ph][b][r, s] = vals[b]

            for b in range(BATCH):
                pltpu.async_copy(xb[ph][b], out_hbm.at[b, rows(ci)],
                                 so[ph][b])
        return 0

    lax.fori_loop(0, NCHUNKS // 2, outer, 0)

    for b in range(BATCH):
        pltpu.make_async_copy(xb[1][b], out_hbm.at[0, rows(0)],
                              so[1][b]).wait()


def kernel(x, pos_table):
    return _sc_pos_add(x, pos_table)
